# baseline (speedup 1.0000x reference)
import jax
import jax.numpy as jnp
from jax.experimental import pallas as pl

N = 10000
E = 160000
H = 128
NH = 8
DH = H // NH
L = 3
NRBF = 50
MAXZ = 16
CUT_LO = 0.0
CUT_HI = 5.0


def _silu(x):
    return x * jax.nn.sigmoid(x)


def _cosine_cutoff(d):
    return 0.5 * (jnp.cos(d * jnp.pi / CUT_HI) + 1.0) * (d < CUT_HI)


def _expnorm_rbf(d, means, betas):
    alpha = 5.0 / (CUT_HI - CUT_LO)
    return jnp.exp(-betas[None, :] * (jnp.exp(alpha * (CUT_LO - d[:, None])) - means[None, :]) ** 2)


def _add_kernel(a_ref, b_ref, o_ref):
    o_ref[...] = a_ref[...] + b_ref[...]


def _pallas_add(a, b):
    return pl.pallas_call(
        _add_kernel,
        out_shape=jax.ShapeDtypeStruct(a.shape, a.dtype),
    )(a, b)


def kernel(z, pos, edge_index, emb, ne_emb, ne_dproj_w, ne_dproj_b, ne_comb_w, ne_comb_b, rbf_means, rbf_betas, conv_lin1_w, filt1_w, filt1_b, filt2_w, filt2_b, q_w, q_b, k_w, k_b, v_w, v_b, o_w, o_b, blk_w, blk_b):
    src, dst = edge_index[0], edge_index[1]
    x = emb[z]
    diff = pos[src] - pos[dst]
    edge_weight = jnp.sqrt(jnp.sum(diff * diff, axis=-1) + 1e-12)
    edge_attr = _expnorm_rbf(edge_weight, rbf_means, rbf_betas)
    C = _cosine_cutoff(edge_weight)
    Wn = (edge_attr @ ne_dproj_w + ne_dproj_b) * C[:, None]
    agg = jnp.zeros((N, H), jnp.float32).at[dst].add(ne_emb[z][src] * Wn)
    x = jnp.concatenate([x, agg], axis=-1) @ ne_comb_w + ne_comb_b
    seg = dst * MAXZ + z[src]
    cnt = jax.ops.segment_sum(jnp.ones((E,), jnp.float32), seg, num_segments=N * MAXZ).reshape(N, MAXZ)
    present = cnt > 0
    for l in range(L):
        Wf = (_silu(edge_attr @ filt1_w[l] + filt1_b[l]) @ filt2_w[l] + filt2_b[l]) * C[:, None]
        msg = (x @ conv_lin1_w[l])[src] * Wf
        y = jax.ops.segment_sum(msg, seg, num_segments=N * MAXZ).reshape(N, MAXZ, H)
        q = (y @ q_w[l] + q_b[l]).reshape(N, MAXZ, NH, DH)
        k = (y @ k_w[l] + k_b[l]).reshape(N, MAXZ, NH, DH)
        v = (y @ v_w[l] + v_b[l]).reshape(N, MAXZ, NH, DH)
        attn = jnp.einsum('nmhd,nkhd->nhmk', q, k)
        attn = jnp.where(present[:, None, None, :], attn, jnp.float32(-1e9))
        p = jax.nn.softmax(attn, axis=-1)
        out = jnp.einsum('nhmk,nkhd->nmhd', p, v).reshape(N, MAXZ, H)
        out = (out @ o_w[l] + o_b[l]) * present[:, :, None].astype(jnp.float32)
        yn = _silu(jnp.sum(out, axis=1))
        x = _pallas_add(x, yn @ blk_w[l] + blk_b[l])
    return x


# speedup vs baseline: 1.3895x; 1.3895x over previous
import functools

import jax
import jax.numpy as jnp
from jax import lax
from jax.experimental import pallas as pl
from jax.experimental.pallas import tpu as pltpu
from jax.experimental.pallas import tpu_sc as plsc

N = 10000
E = 160000
H = 128
NH = 8
DH = H // NH
L = 3
NRBF = 50
MAXZ = 16
CUT_HI = 5.0

NB = 20
SEGB = 8192
NW = 32
CH = 5008
EPAD = 160256
EALLOC = 165120

_MESH = plsc.VectorSubcoreMesh(
    core_axis_name="c", subcore_axis_name="s", num_cores=2, num_subcores=16)


def _cv(ref, i):
    return ref[pl.ds(i, 16)][0]


def _silu(x):
    return x * jax.nn.sigmoid(x)


def _cosine_cutoff(d):
    return 0.5 * (jnp.cos(d * jnp.pi / CUT_HI) + 1.0) * (d < CUT_HI)


def _expnorm_rbf(d, means, betas):
    return jnp.exp(-betas[None, :] * (jnp.exp(-d[:, None]) - means[None, :]) ** 2)


def _prep_body(srce_hbm, dste_hbm, posf_hbm, z_hbm, d2_hbm, seg_hbm, cnt_hbm,
               posf_v, z_v, src_v, dst_v, d2_v, segv_v, cw_v, sm):
    c = lax.axis_index("c")
    s = lax.axis_index("s")
    w = s * 2 + c
    start = w * CH
    nume = jnp.where(w == NW - 1, E - (NW - 1) * CH, CH)
    nv = nume // 16

    pltpu.sync_copy(posf_hbm, posf_v)
    pltpu.sync_copy(z_hbm, z_v)
    pltpu.sync_copy(srce_hbm.at[pl.ds(start, CH)], src_v)
    pltpu.sync_copy(dste_hbm.at[pl.ds(start, CH)], dst_v)

    lanes = lax.iota(jnp.int32, 16)

    def vec_body(i, carry):
        cv0, cv1 = carry
        srcv = src_v[pl.ds(i * 16, 16)]
        dstv = dst_v[pl.ds(i * 16, 16)]
        s3 = srcv * 3
        t3 = dstv * 3
        dx = plsc.load_gather(posf_v, [s3]) - plsc.load_gather(posf_v, [t3])
        dy = plsc.load_gather(posf_v, [s3 + 1]) - plsc.load_gather(posf_v, [t3 + 1])
        dz = plsc.load_gather(posf_v, [s3 + 2]) - plsc.load_gather(posf_v, [t3 + 2])
        d2 = dx * dx + dy * dy + dz * dz
        zsv = plsc.load_gather(z_v, [srcv])
        d2_v[pl.ds(i * 16, 16)] = d2
        segv_v[pl.ds(i * 16, 16)] = dstv * MAXZ + zsv
        bv = lax.shift_right_logical(dstv, 9)
        for b in range(16):
            cv0 = cv0 + jnp.sum((bv == b).astype(jnp.int32)) * (lanes == b).astype(jnp.int32)
        for b in range(16, NB):
            cv1 = cv1 + jnp.sum((bv == b).astype(jnp.int32)) * (lanes == (b - 16)).astype(jnp.int32)
        return cv0, cv1

    zv16 = jnp.zeros((16,), jnp.int32)
    cv0, cv1 = lax.fori_loop(0, nv, vec_body, (zv16, zv16))
    pad8 = lambda v: lax.shift_left(lax.shift_right_logical(v + 7, 3), 3)
    cw_v[pl.ds(0, 16)] = pad8(cv0)
    cw_v[pl.ds(16, 16)] = jnp.where(lanes < NB - 16, pad8(cv1), 0)
    pltpu.sync_copy(cw_v, cnt_hbm.at[w])
    pltpu.sync_copy(d2_v, d2_hbm.at[pl.ds(start, CH)])
    pltpu.sync_copy(segv_v, seg_hbm.at[pl.ds(start, CH)])


@jax.jit
def _sc_prep(srcp, dstp, posf, z):
    return pl.kernel(
        _prep_body,
        out_type=(
            jax.ShapeDtypeStruct((EPAD,), jnp.float32),
            jax.ShapeDtypeStruct((EPAD,), jnp.int32),
            jax.ShapeDtypeStruct((NW, 32), jnp.int32),
        ),
        mesh=_MESH,
        scratch_types=[
            pltpu.VMEM((N * 3,), jnp.float32),
            pltpu.VMEM((N,), jnp.int32),
            pltpu.VMEM((CH,), jnp.int32),
            pltpu.VMEM((CH,), jnp.int32),
            pltpu.VMEM((CH,), jnp.float32),
            pltpu.VMEM((CH,), jnp.int32),
            pltpu.VMEM((32,), jnp.int32),
            pltpu.SMEM((32,), jnp.int32),
        ],
        compiler_params=pltpu.CompilerParams(needs_layout_passes=False),
    )(srcp, dstp, posf, z)


def _place_body(srce_hbm, dste_hbm, sege_hbm, cnts_hbm,
                srcs_hbm, segs_hbm, eids_hbm, cntp_hbm,
                src_v, dst_v, seg_v, bsrc_v, bseg_v, beid_v, cnts_v, cnt_loc,
                sm):
    c = lax.axis_index("c")
    s = lax.axis_index("s")
    w = s * 2 + c
    start = w * CH
    nume = jnp.where(w == NW - 1, E - (NW - 1) * CH, CH)
    nv = nume // 16
    lanes = lax.iota(jnp.int32, 16)

    pltpu.sync_copy(srce_hbm.at[pl.ds(start, CH)], src_v)
    pltpu.sync_copy(dste_hbm.at[pl.ds(start, CH)], dst_v)
    pltpu.sync_copy(sege_hbm.at[pl.ds(start, CH)], seg_v)
    pltpu.sync_copy(cnts_hbm, cnts_v.at[pl.ds(0, NW * 32)])

    loc = jnp.int32(0)
    gb = jnp.int32(0)
    for b in range(NB):
        sm[b] = loc
        sm[40 + b] = loc
        loc = loc + _cv(cnts_v, w * 32 + b)
        part = lax.fori_loop(0, w, lambda wp, a: a + _cv(cnts_v, wp * 32 + b), jnp.int32(0))
        tot = lax.fori_loop(0, NW, lambda wp, a: a + _cv(cnts_v, wp * 32 + b), jnp.int32(0))
        sm[20 + b] = gb + part
        gb = gb + tot

    def vec_body(i, _):
        srcv = src_v[pl.ds(i * 16, 16)]
        dstv = dst_v[pl.ds(i * 16, 16)]
        segv = seg_v[pl.ds(i * 16, 16)]
        eidv = start + i * 16 + lanes
        bv = lax.shift_right_logical(dstv, 9)
        for b in range(NB):
            m = bv == b
            off = sm[40 + b]
            plsc.store_compressed(bsrc_v.at[pl.ds(off, 16)], srcv, mask=m)
            plsc.store_compressed(bseg_v.at[pl.ds(off, 16)], segv, mask=m)
            plsc.store_compressed(beid_v.at[pl.ds(off, 16)], eidv, mask=m)
            sm[40 + b] = off + jnp.sum(m.astype(jnp.int32))
        return _

    lax.fori_loop(0, nv, vec_body, None)

    z16 = jnp.zeros((16,), jnp.int32)
    for b in range(NB):
        cur = sm[40 + b]
        end = sm[b] + _cv(cnts_v, w * 32 + b)
        m = lanes < (end - cur)
        plsc.store_compressed(bsrc_v.at[pl.ds(cur, 16)], z16, mask=m)
        plsc.store_compressed(bseg_v.at[pl.ds(cur, 16)],
                              jnp.full((16,), b * SEGB + SEGB, jnp.int32), mask=m)
        plsc.store_compressed(beid_v.at[pl.ds(cur, 16)], z16, mask=m)

    for b in range(NB):
        lo = sm[b]
        gbase = sm[20 + b]
        pcnt = _cv(cnts_v, w * 32 + b)
        n128 = lax.shift_right_logical(pcnt, 7)
        n8 = lax.shift_right_logical(pcnt - n128 * 128, 3)

        def big(j, _, lo=lo, gbase=gbase):
            go = pl.multiple_of(gbase + j * 128, 8)
            lo8 = pl.multiple_of(lo + j * 128, 8)
            pltpu.sync_copy(bsrc_v.at[pl.ds(lo8, 128)], srcs_hbm.at[pl.ds(go, 128)])
            pltpu.sync_copy(bseg_v.at[pl.ds(lo8, 128)], segs_hbm.at[pl.ds(go, 128)])
            pltpu.sync_copy(beid_v.at[pl.ds(lo8, 128)], eids_hbm.at[pl.ds(go, 128)])
            return _

        def small(j, _, lo=lo, gbase=gbase, n128=n128):
            o = n128 * 128 + j * 8
            go = pl.multiple_of(gbase + o, 8)
            lo8 = pl.multiple_of(lo + o, 8)
            pltpu.sync_copy(bsrc_v.at[pl.ds(lo8, 8)], srcs_hbm.at[pl.ds(go, 8)])
            pltpu.sync_copy(bseg_v.at[pl.ds(lo8, 8)], segs_hbm.at[pl.ds(go, 8)])
            pltpu.sync_copy(beid_v.at[pl.ds(lo8, 8)], eids_hbm.at[pl.ds(go, 8)])
            return _

        lax.fori_loop(0, n128, big, None)
        lax.fori_loop(0, n8, small, None)

    ones16 = jnp.ones((16,), jnp.int32)
    for b in range(NB):
        lo = sm[b]
        pcnt = _cv(cnts_v, w * 32 + b)

        def zero(j, _):
            cnt_loc[pl.ds(j * 16, 16)] = z16
            return _

        lax.fori_loop(0, 513, zero, None)

        def count(j, _, lo=lo, pcnt=pcnt):
            sv = bseg_v[pl.ds(lo + j * 16, 16)]
            idx = sv - b * SEGB
            m = lanes < (pcnt - j * 16)
            plsc.addupdate_scatter(cnt_loc, [idx], ones16, mask=m)
            return _

        lax.fori_loop(0, lax.shift_right_logical(pcnt + 15, 4), count, None)
        pltpu.sync_copy(cnt_loc.at[pl.ds(0, 8200)], cntp_hbm.at[pl.ds(pl.multiple_of(w * (NB * 8200) + b * 8200, 8), 8200)])


@jax.jit
def _sc_place(srcp, dstp, segp, cnts):
    return pl.kernel(
        _place_body,
        out_type=(
            jax.ShapeDtypeStruct((EALLOC,), jnp.int32),
            jax.ShapeDtypeStruct((EALLOC,), jnp.int32),
            jax.ShapeDtypeStruct((EALLOC,), jnp.int32),
            jax.ShapeDtypeStruct((NW * NB * 8200,), jnp.int32),
        ),
        mesh=_MESH,
        scratch_types=[
            pltpu.VMEM((CH,), jnp.int32),
            pltpu.VMEM((CH,), jnp.int32),
            pltpu.VMEM((CH,), jnp.int32),
            pltpu.VMEM((5184,), jnp.int32),
            pltpu.VMEM((5184,), jnp.int32),
            pltpu.VMEM((5184,), jnp.int32),
            pltpu.VMEM((NW * 32 + 16,), jnp.int32),
            pltpu.VMEM((8208,), jnp.int32),
            pltpu.SMEM((64,), jnp.int32),
        ],
        compiler_params=pltpu.CompilerParams(needs_layout_passes=False),
    )(srcp, dstp, segp, cnts)


def _make_segsum(with_xc):
    def body(*refs):
        if with_xc:
            (wf_hbm, xc_hbm, srcs_hbm, segs_hbm, eids_hbm, cnts_hbm, y_hbm,
             acc_sh, zb_v, rows_v, xcr_v, eid_v, srci_v, idx_v, cnts_v, sm, sem) = refs
        else:
            (wf_hbm, srcs_hbm, segs_hbm, eids_hbm, cnts_hbm, y_hbm,
             acc_sh, zb_v, rows_v, eid_v, srci_v, idx_v, cnts_v, sm, sem) = refs
        c = lax.axis_index("c")
        s = lax.axis_index("s")
        lanes = lax.iota(jnp.int32, 16)
        pltpu.sync_copy(cnts_hbm, cnts_v.at[pl.ds(0, NW * 32)])
        gb = jnp.int32(0)
        for b in range(NB):
            sm[b] = gb
            gb = gb + lax.fori_loop(0, NW, lambda wp, a: a + _cv(cnts_v, wp * 32 + b), jnp.int32(0))
        sm[NB] = gb

        zf = jnp.zeros((16,), jnp.float32)

        def zrow(i, _):
            for cc in range(8):
                zb_v[i, pl.ds(cc * 16, 16)] = zf
            return _

        lax.fori_loop(0, 128, zrow, None)

        for j in range(NB // 2):
            b = 2 * j + c
            A = sm[b]
            Bnd = sm[b + 1]
            ln = Bnd - A
            for k in range(4):
                pltpu.sync_copy(zb_v, acc_sh.at[pl.ds(s * 512 + k * 128, 128)])
            plsc.subcore_barrier()
            nch = lax.shift_right_logical(ln + 127, 7)
            my = lax.shift_right_logical(nch - s + 15, 4)

            def chunk(jj, _, A=A, Bnd=Bnd, b=b):
                st = pl.multiple_of(A + (s + jj * 16) * 128, 8)
                pltpu.sync_copy(eids_hbm.at[pl.ds(st, 128)], eid_v)
                pltpu.sync_copy(segs_hbm.at[pl.ds(st, 128)], idx_v)
                if with_xc:
                    pltpu.sync_copy(srcs_hbm.at[pl.ds(st, 128)], srci_v)
                for t in range(8):
                    pos = st + t * 16 + lanes
                    ok = pos < Bnd
                    ev = eid_v[pl.ds(t * 16, 16)]
                    eid_v[pl.ds(t * 16, 16)] = jnp.where(ok, ev, 0)
                    sv = idx_v[pl.ds(t * 16, 16)]
                    idx_v[pl.ds(t * 16, 16)] = jnp.where(ok, sv - b * SEGB, SEGB)
                    if with_xc:
                        rv = srci_v[pl.ds(t * 16, 16)]
                        srci_v[pl.ds(t * 16, 16)] = jnp.where(ok, rv, 0)
                pltpu.async_copy(wf_hbm.at[eid_v], rows_v, sem).wait()
                if with_xc:
                    pltpu.async_copy(xc_hbm.at[srci_v], xcr_v, sem).wait()

                    def mul(r, _):
                        for cc in range(8):
                            rows_v[r, pl.ds(cc * 16, 16)] = (
                                rows_v[r, pl.ds(cc * 16, 16)] * xcr_v[r, pl.ds(cc * 16, 16)])
                        return _

                    lax.fori_loop(0, 128, mul, None, unroll=4)
                pltpu.sync_copy(rows_v, acc_sh.at[idx_v], add=True)
                return _

            lax.fori_loop(0, my, chunk, None)
            plsc.subcore_barrier()
            for k in range(4):
                pltpu.sync_copy(acc_sh.at[pl.ds(s * 512 + k * 128, 128)],
                                y_hbm.at[pl.ds(pl.multiple_of(b * SEGB + s * 512 + k * 128, 8), 128)])
            plsc.subcore_barrier()

    return body


def _segsum_call(with_xc):
    scratch = [
        pltpu.VMEM_SHARED((SEGB + 128, H), jnp.float32),
        pltpu.VMEM((128, H), jnp.float32),
        pltpu.VMEM((128, H), jnp.float32),
    ]
    if with_xc:
        scratch.append(pltpu.VMEM((128, H), jnp.float32))
    scratch += [
        pltpu.VMEM((128,), jnp.int32),
        pltpu.VMEM((128,), jnp.int32),
        pltpu.VMEM((128,), jnp.int32),
        pltpu.VMEM((NW * 32 + 16,), jnp.int32),
        pltpu.SMEM((64,), jnp.int32),
        pltpu.SemaphoreType.DMA,
    ]
    return functools.partial(
        pl.kernel,
        _make_segsum(with_xc),
        out_type=jax.ShapeDtypeStruct((NB * SEGB, H), jnp.float32),
        mesh=_MESH,
        scratch_types=scratch,
        compiler_params=pltpu.CompilerParams(needs_layout_passes=False),
    )()


@jax.jit
def _sc_segsum_plain(wf, srcs, segs, eids, cnts):
    return _segsum_call(False)(wf, srcs, segs, eids, cnts)


@jax.jit
def _sc_segsum_mul(wf, xc, srcs, segs, eids, cnts):
    return _segsum_call(True)(wf, xc, srcs, segs, eids, cnts)


def kernel(z, pos, edge_index, emb, ne_emb, ne_dproj_w, ne_dproj_b, ne_comb_w, ne_comb_b, rbf_means, rbf_betas, conv_lin1_w, filt1_w, filt1_b, filt2_w, filt2_b, q_w, q_b, k_w, k_b, v_w, v_b, o_w, o_b, blk_w, blk_b):
    ei_p = jnp.pad(edge_index.astype(jnp.int32), ((0, 0), (0, EPAD - E)))
    posf = pos.reshape(-1)
    d2p, segp, cnts = _sc_prep(ei_p[0], ei_p[1], posf, z.astype(jnp.int32))
    cnts1d = cnts.reshape(-1)
    srcs, segs, eids, cntp = _sc_place(ei_p[0], ei_p[1], segp, cnts1d)
    d2e = d2p[:E]

    edge_weight = jnp.sqrt(d2e + 1e-12)
    edge_attr = _expnorm_rbf(edge_weight, rbf_means, rbf_betas)
    C = _cosine_cutoff(edge_weight)
    x = emb[z]
    Wn = (edge_attr @ ne_dproj_w + ne_dproj_b) * C[:, None]
    wt = _sc_segsum_plain(Wn, srcs, segs, eids, cnts1d)
    agg = jnp.einsum('nmh,mh->nh', wt[:N * MAXZ].reshape(N, MAXZ, H), ne_emb)
    x = jnp.concatenate([x, agg], axis=-1) @ ne_comb_w + ne_comb_b
    cnt = cntp.reshape(NW, NB * 8200).sum(axis=0).reshape(NB, 8200)[:, :SEGB].reshape(-1)[:N * MAXZ].reshape(N, MAXZ)
    present = cnt > 0
    for l in range(L):
        Wf = (_silu(edge_attr @ filt1_w[l] + filt1_b[l]) @ filt2_w[l] + filt2_b[l]) * C[:, None]
        xc = x @ conv_lin1_w[l]
        y = _sc_segsum_mul(Wf, xc, srcs, segs, eids, cnts1d)[:N * MAXZ].reshape(N, MAXZ, H)
        q = (y @ q_w[l] + q_b[l]).reshape(N, MAXZ, NH, DH)
        k = (y @ k_w[l] + k_b[l]).reshape(N, MAXZ, NH, DH)
        v = (y @ v_w[l] + v_b[l]).reshape(N, MAXZ, NH, DH)
        attn = jnp.einsum('nmhd,nkhd->nhmk', q, k)
        attn = jnp.where(present[:, None, None, :], attn, jnp.float32(-1e9))
        p = jax.nn.softmax(attn, axis=-1)
        out = jnp.einsum('nhmk,nkhd->nmhd', p, v).reshape(N, MAXZ, H)
        out = (out @ o_w[l] + o_b[l]) * present[:, :, None].astype(jnp.float32)
        yn = _silu(jnp.sum(out, axis=1))
        x = x + (yn @ blk_w[l] + blk_b[l])
    return x
